# Initial kernel scaffold; baseline (speedup 1.0000x reference)
import jax
import jax.numpy as jnp
from jax.experimental import pallas as pl


def kernel(ff, edge_index, W_edge, b_edge, W_attn, b_attn, W_node, b_node):
    raise NotImplementedError("write your pallas kernel here")


# speedup vs baseline: 2.2068x; 2.2068x over previous
import functools

import jax
import jax.numpy as jnp
from jax import lax
from jax.experimental import pallas as pl
from jax.experimental.pallas import tpu as pltpu
from jax.experimental.pallas import tpu_sc as plsc

NC = 2
NS = 16
NW = NC * NS
L = 16


def _mm1_body(ff_ref, w_ref, b_ref, p_ref, q_ref, r_ref):
    y = jnp.dot(ff_ref[...], w_ref[...], preferred_element_type=jnp.float32)
    y = y + b_ref[...]
    d = p_ref.shape[1]
    p_ref[...] = y[:, :d]
    q_ref[...] = y[:, d:2 * d]
    r_ref[...] = y[:, 2 * d:]


def _mm2_body(u_ref, den_ref, r_ref, w_ref, o_ref):
    agg = u_ref[...] / (den_ref[:, :1] + 1e-9)
    y = jnp.dot(agg, w_ref[...], preferred_element_type=jnp.float32)
    o_ref[...] = jnp.maximum(y + r_ref[...], 0.0)


def _make_sc_kernel(n, e, d, rows_per, blk):
    n_pad = NW * rows_per
    nchunk = d // L
    nscan = blk // L
    nblk = e // blk

    mesh = plsc.VectorSubcoreMesh(core_axis_name="c", subcore_axis_name="s")

    def body(p_hbm, q_hbm, src_hbm, dst_hbm, wa_hbm, ba_hbm,
             u_hbm, den_hbm,
             acc_u, acc_d, wa_v, ba_v, srcb, dstb, csrc, cdst,
             prows, qrows, sem1, sem2):
        wid = lax.axis_index("s") * NC + lax.axis_index("c")
        base = wid * rows_per

        def zrow(r, _):
            def zch(ch, _):
                acc_u[r, pl.ds(ch * L, L)] = jnp.zeros((L,), jnp.float32)
                return 0
            lax.fori_loop(0, nchunk, zch, 0)
            acc_d[pl.ds(r * L, L)] = jnp.zeros((L,), jnp.float32)
            return 0
        lax.fori_loop(0, rows_per, zrow, 0)

        def zcomp(i, _):
            csrc[pl.ds(i * L, L)] = jnp.zeros((L,), jnp.int32)
            cdst[pl.ds(i * L, L)] = jnp.full((L,), base, jnp.int32)
            return 0
        lax.fori_loop(0, (blk + L) // L, zcomp, 0)

        pltpu.sync_copy(wa_hbm, wa_v)
        pltpu.sync_copy(ba_hbm, ba_v)
        ba = ba_v[...]
        onehot0 = (lax.iota(jnp.int32, L) == 0).astype(jnp.float32)

        def do_block(b, _):
            off = b * blk
            pltpu.sync_copy(src_hbm.at[pl.ds(off, blk)], srcb)
            pltpu.sync_copy(dst_hbm.at[pl.ds(off, blk)], dstb)

            def scan(i, cnt):
                sv = srcb[pl.ds(i * L, L)]
                dv = dstb[pl.ds(i * L, L)]
                msk = (dv >= base) & (dv < base + rows_per)
                plsc.store_compressed(csrc.at[pl.ds(cnt, L)], sv, mask=msk)
                plsc.store_compressed(cdst.at[pl.ds(cnt, L)], dv, mask=msk)
                return cnt + jnp.sum(msk.astype(jnp.int32))
            cnt = lax.fori_loop(0, nscan, scan, 0)

            ngroups = (cnt + (L - 1)) // L

            def group(g, _):
                c1 = pltpu.async_copy(
                    p_hbm.at[csrc.at[pl.ds(g * L, L)]], prows, sem1)
                c2 = pltpu.async_copy(
                    q_hbm.at[cdst.at[pl.ds(g * L, L)]], qrows, sem2)
                c1.wait()
                c2.wait()
                rvec = cdst[pl.ds(g * L, L)] - base
                for i in range(L):
                    j = g * L + i
                    rloc = rvec[i]

                    def dot_ch(ch, acc):
                        o = ch * L
                        m = jnp.maximum(prows[i, pl.ds(o, L)]
                                        + qrows[i, pl.ds(o, L)], 0.0)
                        prows[i, pl.ds(o, L)] = m
                        return acc + m * wa_v[pl.ds(o, L)]
                    acc = lax.fori_loop(0, nchunk, dot_ch,
                                        jnp.zeros((L,), jnp.float32))
                    logit = jnp.sum(acc)
                    valid = jnp.where(j < cnt, 1.0, 0.0)
                    exv = jnp.exp(jnp.full((L,), logit, jnp.float32) + ba)
                    exv = exv * valid

                    def acc_ch(ch, _):
                        o = ch * L
                        plsc.addupdate(acc_u.at[rloc, pl.ds(o, L)],
                                       exv * prows[i, pl.ds(o, L)])
                        return 0
                    lax.fori_loop(0, nchunk, acc_ch, 0)
                    plsc.addupdate(acc_d.at[pl.ds(rloc * L, L)], exv * onehot0)
                return 0
            lax.fori_loop(0, ngroups, group, 0)
            return 0
        lax.fori_loop(0, nblk, do_block, 0)

        pltpu.sync_copy(acc_u, u_hbm.at[pl.ds(base, rows_per)])
        pltpu.sync_copy(acc_d, den_hbm.at[pl.ds(base * L, rows_per * L)])

    return pl.kernel(
        body,
        out_type=[jax.ShapeDtypeStruct((n_pad, d), jnp.float32),
                  jax.ShapeDtypeStruct((n_pad * L,), jnp.float32)],
        mesh=mesh,
        compiler_params=pltpu.CompilerParams(needs_layout_passes=False),
        scratch_types=[
            pltpu.VMEM((rows_per, d), jnp.float32),
            pltpu.VMEM((rows_per * L,), jnp.float32),
            pltpu.VMEM((d,), jnp.float32),
            pltpu.VMEM((L,), jnp.float32),
            pltpu.VMEM((blk,), jnp.int32),
            pltpu.VMEM((blk,), jnp.int32),
            pltpu.VMEM((blk + L,), jnp.int32),
            pltpu.VMEM((blk + L,), jnp.int32),
            pltpu.VMEM((L, d), jnp.float32),
            pltpu.VMEM((L, d), jnp.float32),
            pltpu.SemaphoreType.DMA,
            pltpu.SemaphoreType.DMA,
        ],
    )


def kernel(ff, edge_index, W_edge, b_edge, W_attn, b_attn, W_node, b_node):
    n, d = ff.shape
    e = edge_index.shape[1]
    rows_per = (-(-n // NW) + 7) // 8 * 8
    blk = 1600
    assert e % blk == 0 and d % L == 0 and blk % L == 0

    w_cat = jnp.concatenate(
        [W_edge[:d, :], W_edge[d:, :], W_node[:d, :]], axis=1)
    b_cat = jnp.concatenate(
        [jnp.zeros((d,), jnp.float32), b_edge, b_node])[None, :]
    rb = 400
    grid = (n // rb,)
    P, Q, R = pl.pallas_call(
        _mm1_body,
        grid=grid,
        in_specs=[
            pl.BlockSpec((rb, d), lambda i: (i, 0)),
            pl.BlockSpec((d, 3 * d), lambda i: (0, 0)),
            pl.BlockSpec((1, 3 * d), lambda i: (0, 0)),
        ],
        out_specs=[
            pl.BlockSpec((rb, d), lambda i: (i, 0)),
            pl.BlockSpec((rb, d), lambda i: (i, 0)),
            pl.BlockSpec((rb, d), lambda i: (i, 0)),
        ],
        out_shape=[jax.ShapeDtypeStruct((n, d), jnp.float32)] * 3,
    )(ff, w_cat, b_cat)

    sc = _make_sc_kernel(n, e, d, rows_per, blk)
    wa = W_attn[:, 0]
    ba16 = jnp.full((L,), b_attn[0], jnp.float32)
    U, den_flat = sc(P, Q, edge_index[0], edge_index[1], wa, ba16)
    den = den_flat.reshape(-1, L)

    out = pl.pallas_call(
        _mm2_body,
        grid=grid,
        in_specs=[
            pl.BlockSpec((rb, d), lambda i: (i, 0)),
            pl.BlockSpec((rb, L), lambda i: (i, 0)),
            pl.BlockSpec((rb, d), lambda i: (i, 0)),
            pl.BlockSpec((d, d), lambda i: (0, 0)),
        ],
        out_specs=pl.BlockSpec((rb, d), lambda i: (i, 0)),
        out_shape=jax.ShapeDtypeStruct((n, d), jnp.float32),
    )(U[:n], den[:n], R, W_node[d:, :])
    return out


# speedup vs baseline: 4.6045x; 2.0865x over previous
import functools

import jax
import jax.numpy as jnp
from jax import lax
from jax.experimental import pallas as pl
from jax.experimental.pallas import tpu as pltpu
from jax.experimental.pallas import tpu_sc as plsc

NC = 2
NS = 16
NW = NC * NS
L = 16


def _mm1_body(ff_ref, w_ref, b_ref, p_ref, q_ref, r_ref):
    y = jnp.dot(ff_ref[...], w_ref[...], preferred_element_type=jnp.float32)
    y = y + b_ref[...]
    d = p_ref.shape[1]
    p_ref[...] = y[:, :d].astype(jnp.bfloat16)
    q_ref[...] = y[:, d:2 * d].astype(jnp.bfloat16)
    r_ref[...] = y[:, 2 * d:]


def _mm2_body(u_ref, den_ref, r_ref, w_ref, o_ref):
    agg = u_ref[...] / (den_ref[:, :1] + 1e-9)
    y = jnp.dot(agg, w_ref[...], preferred_element_type=jnp.float32)
    o_ref[...] = jnp.maximum(y + r_ref[...], 0.0)


def _make_sc_kernel(n, e, d, rows_per, blk):
    n_pad = NW * rows_per
    nchunk = d // L
    nscan = blk // L
    nblk = e // blk

    mesh = plsc.VectorSubcoreMesh(core_axis_name="c", subcore_axis_name="s")

    def body(p_hbm, q_hbm, src_hbm, dst_hbm, wa_hbm, ba_hbm,
             u_hbm, den_hbm,
             acc_u, acc_d, wa_v, ba_v, srcb, dstb, srcb2, dstb2, csrc, cdst,
             prows, qrows, prows2, qrows2, sem1, sem2, sem3, sem4, sem5, sem6):
        wid = lax.axis_index("s") * NC + lax.axis_index("c")
        base = wid * rows_per

        def zrow(r, _):
            def zch(ch, _):
                acc_u[r, pl.ds(ch * L, L)] = jnp.zeros((L,), jnp.float32)
                return 0
            lax.fori_loop(0, nchunk, zch, 0)
            acc_d[pl.ds(r * L, L)] = jnp.zeros((L,), jnp.float32)
            return 0
        lax.fori_loop(0, rows_per, zrow, 0)

        def zcomp(i, _):
            csrc[pl.ds(i * L, L)] = jnp.zeros((L,), jnp.int32)
            cdst[pl.ds(i * L, L)] = jnp.full((L,), base, jnp.int32)
            return 0
        lax.fori_loop(0, (blk + L) // L, zcomp, 0)

        pltpu.sync_copy(wa_hbm, wa_v)
        pltpu.sync_copy(ba_hbm, ba_v)
        ba = ba_v[...]
        onehot0 = (lax.iota(jnp.int32, L) == 0).astype(jnp.float32)

        def fire_ids(b, sb, db, s):
            off = b * blk
            pltpu.async_copy(src_hbm.at[pl.ds(off, blk)], sb, s)
            pltpu.async_copy(dst_hbm.at[pl.ds(off, blk)], db, s)

        def wait_ids(sb, db, s):
            pltpu.make_async_copy(src_hbm.at[pl.ds(0, blk)], sb, s).wait()
            pltpu.make_async_copy(dst_hbm.at[pl.ds(0, blk)], db, s).wait()

        def scan_block(sb, db):
            def scan(i, cnt):
                sv = sb[pl.ds(i * L, L)]
                dv = db[pl.ds(i * L, L)]
                msk = (dv >= base) & (dv < base + rows_per)
                plsc.store_compressed(csrc.at[pl.ds(cnt, L)], sv, mask=msk)
                plsc.store_compressed(cdst.at[pl.ds(cnt, L)], dv, mask=msk)
                return cnt + plsc.all_reduce_population_count(msk)[0]
            return lax.fori_loop(0, nscan, scan, 0, unroll=4)

        def fire_rows(g, pb, qb, s1, s2):
            pltpu.async_copy(p_hbm.at[csrc.at[pl.ds(g * L, L)]], pb, s1)
            pltpu.async_copy(q_hbm.at[cdst.at[pl.ds(g * L, L)]], qb, s2)

        def wait_rows(pb, qb, s1, s2):
            pltpu.make_async_copy(p_hbm.at[csrc.at[pl.ds(0, L)]],
                                  pb, s1).wait()
            pltpu.make_async_copy(q_hbm.at[cdst.at[pl.ds(0, L)]],
                                  qb, s2).wait()

        def process(g, cnt, pb, qb):
            def edge(i, _):
                j = g * L + i
                rloc = cdst[pl.ds(j, L)][0] - base
                ms = []
                acc = jnp.zeros((L,), jnp.float32)
                for chp in range(nchunk // 2):
                    o = chp * L
                    praw = plsc.bitcast(pb[i, pl.ds(o, L)], jnp.bfloat16)
                    qraw = plsc.bitcast(qb[i, pl.ds(o, L)], jnp.bfloat16)
                    pa, pb2 = plsc.unpack(praw,
                                          format=plsc.PackFormat.INTERLEAVED)
                    qa, qb2 = plsc.unpack(qraw,
                                          format=plsc.PackFormat.INTERLEAVED)
                    ma = jnp.maximum(pa + qa, 0.0)
                    mb = jnp.maximum(pb2 + qb2, 0.0)
                    ms.append(ma)
                    ms.append(mb)
                    o16 = chp * 2 * L
                    acc = (acc + ma * wa_v[pl.ds(o16, L)]
                           + mb * wa_v[pl.ds(o16 + L, L)])
                logit = jnp.sum(acc)
                valid = jnp.where(j < cnt, 1.0, 0.0)
                exv = jnp.exp(jnp.full((L,), logit, jnp.float32) + ba)
                exv = exv * valid
                for ch in range(nchunk):
                    plsc.addupdate(acc_u.at[rloc, pl.ds(ch * L, L)],
                                   exv * ms[ch])
                plsc.addupdate(acc_d.at[pl.ds(rloc * L, L)], exv * onehot0)
                return 0
            lax.fori_loop(0, L, edge, 0, unroll=2)

        def process_groups(cnt):
            ngroups = (cnt + (L - 1)) // L

            @pl.when(ngroups > 0)
            def _():
                fire_rows(0, prows, qrows, sem1, sem2)

            def pair(p, _):
                g0 = 2 * p

                @pl.when(g0 + 1 < ngroups)
                def _():
                    fire_rows(g0 + 1, prows2, qrows2, sem3, sem4)
                wait_rows(prows, qrows, sem1, sem2)
                process(g0, cnt, prows, qrows)

                @pl.when(g0 + 2 < ngroups)
                def _():
                    fire_rows(g0 + 2, prows, qrows, sem1, sem2)

                @pl.when(g0 + 1 < ngroups)
                def _():
                    wait_rows(prows2, qrows2, sem3, sem4)
                    process(g0 + 1, cnt, prows2, qrows2)
                return 0
            lax.fori_loop(0, (ngroups + 1) // 2, pair, 0)

        fire_ids(0, srcb, dstb, sem5)

        def blkpair(p, _):
            b0 = 2 * p
            fire_ids(b0 + 1, srcb2, dstb2, sem6)
            wait_ids(srcb, dstb, sem5)
            cnt = scan_block(srcb, dstb)
            process_groups(cnt)

            @pl.when(b0 + 2 < nblk)
            def _():
                fire_ids(b0 + 2, srcb, dstb, sem5)
            wait_ids(srcb2, dstb2, sem6)
            cnt2 = scan_block(srcb2, dstb2)
            process_groups(cnt2)
            return 0
        lax.fori_loop(0, nblk // 2, blkpair, 0)

        pltpu.sync_copy(acc_u, u_hbm.at[pl.ds(base, rows_per)])
        pltpu.sync_copy(acc_d, den_hbm.at[pl.ds(base * L, rows_per * L)])

    return pl.kernel(
        body,
        out_type=[jax.ShapeDtypeStruct((n_pad, d), jnp.float32),
                  jax.ShapeDtypeStruct((n_pad * L,), jnp.float32)],
        mesh=mesh,
        compiler_params=pltpu.CompilerParams(needs_layout_passes=False),
        scratch_types=[
            pltpu.VMEM((rows_per, d), jnp.float32),
            pltpu.VMEM((rows_per * L,), jnp.float32),
            pltpu.VMEM((d,), jnp.float32),
            pltpu.VMEM((L,), jnp.float32),
            pltpu.VMEM((blk,), jnp.int32),
            pltpu.VMEM((blk,), jnp.int32),
            pltpu.VMEM((blk,), jnp.int32),
            pltpu.VMEM((blk,), jnp.int32),
            pltpu.VMEM((blk + L,), jnp.int32),
            pltpu.VMEM((blk + L,), jnp.int32),
            pltpu.VMEM((L, d // 2), jnp.int32),
            pltpu.VMEM((L, d // 2), jnp.int32),
            pltpu.VMEM((L, d // 2), jnp.int32),
            pltpu.VMEM((L, d // 2), jnp.int32),
            pltpu.SemaphoreType.DMA,
            pltpu.SemaphoreType.DMA,
            pltpu.SemaphoreType.DMA,
            pltpu.SemaphoreType.DMA,
            pltpu.SemaphoreType.DMA,
            pltpu.SemaphoreType.DMA,
        ],
    )


def kernel(ff, edge_index, W_edge, b_edge, W_attn, b_attn, W_node, b_node):
    n, d = ff.shape
    e = edge_index.shape[1]
    rows_per = (-(-n // NW) + 7) // 8 * 8
    blk = 1600
    assert e % blk == 0 and d % L == 0 and blk % L == 0

    w_cat = jnp.concatenate(
        [W_edge[:d, :], W_edge[d:, :], W_node[:d, :]], axis=1)
    b_cat = jnp.concatenate(
        [jnp.zeros((d,), jnp.float32), b_edge, b_node])[None, :]
    rb = 400
    grid = (n // rb,)
    P, Q, R = pl.pallas_call(
        _mm1_body,
        grid=grid,
        in_specs=[
            pl.BlockSpec((rb, d), lambda i: (i, 0)),
            pl.BlockSpec((d, 3 * d), lambda i: (0, 0)),
            pl.BlockSpec((1, 3 * d), lambda i: (0, 0)),
        ],
        out_specs=[
            pl.BlockSpec((rb, d), lambda i: (i, 0)),
            pl.BlockSpec((rb, d), lambda i: (i, 0)),
            pl.BlockSpec((rb, d), lambda i: (i, 0)),
        ],
        out_shape=[jax.ShapeDtypeStruct((n, d), jnp.bfloat16),
                   jax.ShapeDtypeStruct((n, d), jnp.bfloat16),
                   jax.ShapeDtypeStruct((n, d), jnp.float32)],
    )(ff, w_cat, b_cat)

    ch = jnp.arange(d) // L
    lane = jnp.arange(d) % L
    perm = 32 * (ch // 2) + 2 * lane + (ch % 2)
    sc = _make_sc_kernel(n, e, d, rows_per, blk)
    wa = W_attn[perm, 0]
    ba16 = jnp.full((L,), b_attn[0], jnp.float32)
    Pb = lax.bitcast_convert_type(P.reshape(n, d // 2, 2), jnp.int32)
    Qb = lax.bitcast_convert_type(Q.reshape(n, d // 2, 2), jnp.int32)
    U, den_flat = sc(Pb, Qb, edge_index[0], edge_index[1], wa, ba16)
    den = den_flat.reshape(-1, L)

    out = pl.pallas_call(
        _mm2_body,
        grid=grid,
        in_specs=[
            pl.BlockSpec((rb, d), lambda i: (i, 0)),
            pl.BlockSpec((rb, L), lambda i: (i, 0)),
            pl.BlockSpec((rb, d), lambda i: (i, 0)),
            pl.BlockSpec((d, d), lambda i: (0, 0)),
        ],
        out_specs=pl.BlockSpec((rb, d), lambda i: (i, 0)),
        out_shape=jax.ShapeDtypeStruct((n, d), jnp.float32),
    )(U[:n], den[:n], R, W_node[d:, :][perm, :])
    return out


# speedup vs baseline: 5.3482x; 1.1615x over previous
import functools

import jax
import jax.numpy as jnp
from jax import lax
from jax.experimental import pallas as pl
from jax.experimental.pallas import tpu as pltpu
from jax.experimental.pallas import tpu_sc as plsc

NC = 2
NS = 16
NW = NC * NS
L = 16


def _mm1_body(ff_ref, w_ref, b_ref, p_ref, q_ref, r_ref):
    y = jnp.dot(ff_ref[...], w_ref[...], preferred_element_type=jnp.float32)
    y = y + b_ref[...]
    d = p_ref.shape[1]
    p_ref[...] = y[:, :d]
    q_ref[...] = y[:, d:2 * d]
    r_ref[...] = y[:, 2 * d:]


def _mm2_body(u_ref, den_ref, r_ref, w_ref, o_ref):
    agg = u_ref[...] / (den_ref[:, :1] + 1e-9)
    y = jnp.dot(agg, w_ref[...], preferred_element_type=jnp.float32)
    o_ref[...] = jnp.maximum(y + r_ref[...], 0.0)


def _make_sc_kernel(n, e, d, rows_per, blk):
    n_pad = NW * rows_per
    nchunk = d // L
    nscan = blk // L
    nblk = e // blk

    mesh = plsc.VectorSubcoreMesh(core_axis_name="c", subcore_axis_name="s")

    def body(p_hbm, q_hbm, src_hbm, dst_hbm, wa_hbm, ba_hbm,
             u_hbm, den_hbm,
             acc_u, acc_d, wa_v, ba_v, srcb, dstb, srcb2, dstb2, csrc, cdst,
             prows, qrows, prows2, qrows2, sem1, sem2, sem3, sem4, sem5, sem6):
        wid = lax.axis_index("s") * NC + lax.axis_index("c")
        base = wid * rows_per

        def zrow(r, _):
            def zch(ch, _):
                acc_u[r, pl.ds(ch * L, L)] = jnp.zeros((L,), jnp.float32)
                return 0
            lax.fori_loop(0, nchunk, zch, 0)
            acc_d[pl.ds(r * L, L)] = jnp.zeros((L,), jnp.float32)
            return 0
        lax.fori_loop(0, rows_per, zrow, 0)

        def zcomp(i, _):
            csrc[pl.ds(i * L, L)] = jnp.zeros((L,), jnp.int32)
            cdst[pl.ds(i * L, L)] = jnp.full((L,), base, jnp.int32)
            return 0
        lax.fori_loop(0, (blk + L) // L, zcomp, 0)

        pltpu.sync_copy(wa_hbm, wa_v)
        pltpu.sync_copy(ba_hbm, ba_v)
        ba = ba_v[...]
        onehot0 = (lax.iota(jnp.int32, L) == 0).astype(jnp.float32)

        def fire_ids(b, sb, db, s):
            off = b * blk
            pltpu.async_copy(src_hbm.at[pl.ds(off, blk)], sb, s)
            pltpu.async_copy(dst_hbm.at[pl.ds(off, blk)], db, s)

        def wait_ids(sb, db, s):
            pltpu.make_async_copy(src_hbm.at[pl.ds(0, blk)], sb, s).wait()
            pltpu.make_async_copy(dst_hbm.at[pl.ds(0, blk)], db, s).wait()

        def scan_block(sb, db):
            def scan(i, cnt):
                sv = sb[pl.ds(i * L, L)]
                dv = db[pl.ds(i * L, L)]
                msk = (dv >= base) & (dv < base + rows_per)
                plsc.store_compressed(csrc.at[pl.ds(cnt, L)], sv, mask=msk)
                plsc.store_compressed(cdst.at[pl.ds(cnt, L)], dv, mask=msk)
                return cnt + plsc.all_reduce_population_count(msk)[0]
            return lax.fori_loop(0, nscan, scan, 0, unroll=4)

        def fire_rows(g, pb, qb, s1, s2):
            pltpu.async_copy(p_hbm.at[csrc.at[pl.ds(g * L, L)]], pb, s1)
            pltpu.async_copy(q_hbm.at[cdst.at[pl.ds(g * L, L)]], qb, s2)

        def wait_rows(pb, qb, s1, s2):
            pltpu.make_async_copy(p_hbm.at[csrc.at[pl.ds(0, L)]],
                                  pb, s1).wait()
            pltpu.make_async_copy(q_hbm.at[cdst.at[pl.ds(0, L)]],
                                  qb, s2).wait()

        def process(g, cnt, pb, qb):
            def edge(i, _):
                j = g * L + i
                rloc = cdst[pl.ds(j, L)][0] - base
                ms = []
                acc = jnp.zeros((L,), jnp.float32)
                for ch in range(nchunk):
                    o = ch * L
                    m = jnp.maximum(pb[i, pl.ds(o, L)]
                                    + qb[i, pl.ds(o, L)], 0.0)
                    ms.append(m)
                    acc = acc + m * wa_v[pl.ds(o, L)]
                logit = jnp.sum(acc)
                valid = jnp.where(j < cnt, 1.0, 0.0)
                exv = jnp.exp(jnp.full((L,), logit, jnp.float32) + ba)
                exv = exv * valid
                for ch in range(nchunk):
                    plsc.addupdate(acc_u.at[rloc, pl.ds(ch * L, L)],
                                   exv * ms[ch])
                plsc.addupdate(acc_d.at[pl.ds(rloc * L, L)], exv * onehot0)
                return 0
            lax.fori_loop(0, L, edge, 0, unroll=2)

        def process_groups(cnt):
            ngroups = (cnt + (L - 1)) // L

            @pl.when(ngroups > 0)
            def _():
                fire_rows(0, prows, qrows, sem1, sem2)

            def pair(p, _):
                g0 = 2 * p

                @pl.when(g0 + 1 < ngroups)
                def _():
                    fire_rows(g0 + 1, prows2, qrows2, sem3, sem4)
                wait_rows(prows, qrows, sem1, sem2)
                process(g0, cnt, prows, qrows)

                @pl.when(g0 + 2 < ngroups)
                def _():
                    fire_rows(g0 + 2, prows, qrows, sem1, sem2)

                @pl.when(g0 + 1 < ngroups)
                def _():
                    wait_rows(prows2, qrows2, sem3, sem4)
                    process(g0 + 1, cnt, prows2, qrows2)
                return 0
            lax.fori_loop(0, (ngroups + 1) // 2, pair, 0)

        fire_ids(0, srcb, dstb, sem5)

        def blkpair(p, _):
            b0 = 2 * p
            fire_ids(b0 + 1, srcb2, dstb2, sem6)
            wait_ids(srcb, dstb, sem5)
            cnt = scan_block(srcb, dstb)
            process_groups(cnt)

            @pl.when(b0 + 2 < nblk)
            def _():
                fire_ids(b0 + 2, srcb, dstb, sem5)
            wait_ids(srcb2, dstb2, sem6)
            cnt2 = scan_block(srcb2, dstb2)
            process_groups(cnt2)
            return 0
        lax.fori_loop(0, nblk // 2, blkpair, 0)

        pltpu.sync_copy(acc_u, u_hbm.at[pl.ds(base, rows_per)])
        pltpu.sync_copy(acc_d, den_hbm.at[pl.ds(base * L, rows_per * L)])

    return pl.kernel(
        body,
        out_type=[jax.ShapeDtypeStruct((n_pad, d), jnp.float32),
                  jax.ShapeDtypeStruct((n_pad * L,), jnp.float32)],
        mesh=mesh,
        compiler_params=pltpu.CompilerParams(needs_layout_passes=False, disable_bounds_checks=True),
        scratch_types=[
            pltpu.VMEM((rows_per, d), jnp.float32),
            pltpu.VMEM((rows_per * L,), jnp.float32),
            pltpu.VMEM((d,), jnp.float32),
            pltpu.VMEM((L,), jnp.float32),
            pltpu.VMEM((blk,), jnp.int32),
            pltpu.VMEM((blk,), jnp.int32),
            pltpu.VMEM((blk,), jnp.int32),
            pltpu.VMEM((blk,), jnp.int32),
            pltpu.VMEM((blk + L,), jnp.int32),
            pltpu.VMEM((blk + L,), jnp.int32),
            pltpu.VMEM((L, d), jnp.float32),
            pltpu.VMEM((L, d), jnp.float32),
            pltpu.VMEM((L, d), jnp.float32),
            pltpu.VMEM((L, d), jnp.float32),
            pltpu.SemaphoreType.DMA,
            pltpu.SemaphoreType.DMA,
            pltpu.SemaphoreType.DMA,
            pltpu.SemaphoreType.DMA,
            pltpu.SemaphoreType.DMA,
            pltpu.SemaphoreType.DMA,
        ],
    )


def kernel(ff, edge_index, W_edge, b_edge, W_attn, b_attn, W_node, b_node):
    n, d = ff.shape
    e = edge_index.shape[1]
    rows_per = (-(-n // NW) + 7) // 8 * 8
    blk = 1600
    assert e % blk == 0 and d % L == 0 and blk % L == 0

    w_cat = jnp.concatenate(
        [W_edge[:d, :], W_edge[d:, :], W_node[:d, :]], axis=1)
    b_cat = jnp.concatenate(
        [jnp.zeros((d,), jnp.float32), b_edge, b_node])[None, :]
    rb = 400
    grid = (n // rb,)
    P, Q, R = pl.pallas_call(
        _mm1_body,
        grid=grid,
        in_specs=[
            pl.BlockSpec((rb, d), lambda i: (i, 0)),
            pl.BlockSpec((d, 3 * d), lambda i: (0, 0)),
            pl.BlockSpec((1, 3 * d), lambda i: (0, 0)),
        ],
        out_specs=[
            pl.BlockSpec((rb, d), lambda i: (i, 0)),
            pl.BlockSpec((rb, d), lambda i: (i, 0)),
            pl.BlockSpec((rb, d), lambda i: (i, 0)),
        ],
        out_shape=[jax.ShapeDtypeStruct((n, d), jnp.float32)] * 3,
    )(ff, w_cat, b_cat)

    sc = _make_sc_kernel(n, e, d, rows_per, blk)
    wa = W_attn[:, 0]
    ba16 = jnp.full((L,), b_attn[0], jnp.float32)
    U, den_flat = sc(P, Q, edge_index[0], edge_index[1], wa, ba16)
    den = den_flat.reshape(-1, L)

    out = pl.pallas_call(
        _mm2_body,
        grid=grid,
        in_specs=[
            pl.BlockSpec((rb, d), lambda i: (i, 0)),
            pl.BlockSpec((rb, L), lambda i: (i, 0)),
            pl.BlockSpec((rb, d), lambda i: (i, 0)),
            pl.BlockSpec((d, d), lambda i: (0, 0)),
        ],
        out_specs=pl.BlockSpec((rb, d), lambda i: (i, 0)),
        out_shape=jax.ShapeDtypeStruct((n, d), jnp.float32),
    )(U[:n], den[:n], R, W_node[d:, :])
    return out


# speedup vs baseline: 6.1534x; 1.1506x over previous
import functools

import jax
import jax.numpy as jnp
from jax import lax
from jax.experimental import pallas as pl
from jax.experimental.pallas import tpu as pltpu
from jax.experimental.pallas import tpu_sc as plsc

NC = 2
NS = 16
NW = NC * NS
L = 16


def _mm1_body(ff_ref, w_ref, b_ref, p_ref, q_ref, r_ref):
    y = jnp.dot(ff_ref[...], w_ref[...], preferred_element_type=jnp.float32)
    y = y + b_ref[...]
    d = p_ref.shape[1]
    p_ref[...] = y[:, :d]
    q_ref[...] = y[:, d:2 * d]
    r_ref[...] = y[:, 2 * d:]


def _mm2_body(u_ref, den_ref, r_ref, w_ref, o_ref):
    agg = u_ref[...] / (den_ref[:, :1] + 1e-9)
    y = jnp.dot(agg, w_ref[...], preferred_element_type=jnp.float32)
    o_ref[...] = jnp.maximum(y + r_ref[...], 0.0)


def _make_sc_kernel(n, e, d, rows_per, blk):
    n_pad = NW * rows_per
    nchunk = d // L
    nscan = blk // L
    nblk = e // blk

    mesh = plsc.VectorSubcoreMesh(core_axis_name="c", subcore_axis_name="s")

    def body(p_hbm, q_hbm, src_hbm, dst_hbm, wa_hbm, ba_hbm,
             u_hbm, den_hbm,
             acc_u, acc_d, wa_v, ba_v, srcb, dstb, srcb2, dstb2, csrc, cdst,
             prows, qrows, prows2, qrows2, sem1, sem2, sem3, sem4, sem5, sem6):
        wid = lax.axis_index("s") * NC + lax.axis_index("c")
        base = wid * rows_per

        def zrow(r, _):
            def zch(ch, _):
                acc_u[r, pl.ds(ch * L, L)] = jnp.zeros((L,), jnp.float32)
                return 0
            lax.fori_loop(0, nchunk, zch, 0)
            acc_d[pl.ds(r * L, L)] = jnp.zeros((L,), jnp.float32)
            return 0
        lax.fori_loop(0, rows_per, zrow, 0)

        def zcomp(i, _):
            csrc[pl.ds(i * L, L)] = jnp.zeros((L,), jnp.int32)
            cdst[pl.ds(i * L, L)] = jnp.full((L,), base, jnp.int32)
            return 0
        lax.fori_loop(0, (2 * blk + 3 * L) // L, zcomp, 0)

        pltpu.sync_copy(wa_hbm, wa_v)
        pltpu.sync_copy(ba_hbm, ba_v)
        ba = ba_v[...]
        onehot0 = (lax.iota(jnp.int32, L) == 0).astype(jnp.float32)

        def fire_ids(b, sb, db, s):
            off = b * blk
            pltpu.async_copy(src_hbm.at[pl.ds(off, blk)], sb, s)
            pltpu.async_copy(dst_hbm.at[pl.ds(off, blk)], db, s)

        def wait_ids(sb, db, s):
            pltpu.make_async_copy(src_hbm.at[pl.ds(0, blk)], sb, s).wait()
            pltpu.make_async_copy(dst_hbm.at[pl.ds(0, blk)], db, s).wait()

        def scan_block(sb, db, cnt0):
            def scan(i, cnt):
                sv = sb[pl.ds(i * L, L)]
                dv = db[pl.ds(i * L, L)]
                msk = (dv >= base) & (dv < base + rows_per)
                plsc.store_compressed(csrc.at[pl.ds(cnt, L)], sv, mask=msk)
                plsc.store_compressed(cdst.at[pl.ds(cnt, L)], dv, mask=msk)
                return cnt + plsc.all_reduce_population_count(msk)[0]
            return lax.fori_loop(0, nscan, scan, cnt0, unroll=4)

        def fire_rows(g, pb, qb, s1, s2):
            pltpu.async_copy(p_hbm.at[csrc.at[pl.ds(g * L, L)]], pb, s1)
            pltpu.async_copy(q_hbm.at[cdst.at[pl.ds(g * L, L)]], qb, s2)

        def wait_rows(pb, qb, s1, s2):
            pltpu.make_async_copy(p_hbm.at[csrc.at[pl.ds(0, L)]],
                                  pb, s1).wait()
            pltpu.make_async_copy(q_hbm.at[cdst.at[pl.ds(0, L)]],
                                  qb, s2).wait()

        def process(g, cnt, pb, qb):
            def edge(i, _):
                j = g * L + i
                rloc = cdst[pl.ds(j, L)][0] - base
                ms = []
                acc = jnp.zeros((L,), jnp.float32)
                for ch in range(nchunk):
                    o = ch * L
                    m = jnp.maximum(pb[i, pl.ds(o, L)]
                                    + qb[i, pl.ds(o, L)], 0.0)
                    ms.append(m)
                    acc = acc + m * wa_v[pl.ds(o, L)]
                logit = jnp.sum(acc)
                valid = jnp.where(j < cnt, 1.0, 0.0)
                exv = jnp.exp(jnp.full((L,), logit, jnp.float32) + ba)
                exv = exv * valid
                for ch in range(nchunk):
                    plsc.addupdate(acc_u.at[rloc, pl.ds(ch * L, L)],
                                   exv * ms[ch])
                plsc.addupdate(acc_d.at[pl.ds(rloc * L, L)], exv * onehot0)
                return 0
            lax.fori_loop(0, L, edge, 0, unroll=2)

        def process_groups(cnt):
            ngroups = (cnt + (L - 1)) // L

            @pl.when(ngroups > 0)
            def _():
                fire_rows(0, prows, qrows, sem1, sem2)

            def pair(p, _):
                g0 = 2 * p

                @pl.when(g0 + 1 < ngroups)
                def _():
                    fire_rows(g0 + 1, prows2, qrows2, sem3, sem4)
                wait_rows(prows, qrows, sem1, sem2)
                process(g0, cnt, prows, qrows)

                @pl.when(g0 + 2 < ngroups)
                def _():
                    fire_rows(g0 + 2, prows, qrows, sem1, sem2)

                @pl.when(g0 + 1 < ngroups)
                def _():
                    wait_rows(prows2, qrows2, sem3, sem4)
                    process(g0 + 1, cnt, prows2, qrows2)
                return 0
            lax.fori_loop(0, (ngroups + 1) // 2, pair, 0)

        fire_ids(0, srcb, dstb, sem5)

        fire_ids(1, srcb2, dstb2, sem6)

        def blkpair(p, _):
            b0 = 2 * p
            wait_ids(srcb, dstb, sem5)
            cnt1 = scan_block(srcb, dstb, 0)
            wait_ids(srcb2, dstb2, sem6)
            cnt = scan_block(srcb2, dstb2, cnt1)

            @pl.when(b0 + 2 < nblk)
            def _():
                fire_ids(b0 + 2, srcb, dstb, sem5)
                fire_ids(b0 + 3, srcb2, dstb2, sem6)
            process_groups(cnt)
            return 0
        lax.fori_loop(0, nblk // 2, blkpair, 0)

        pltpu.sync_copy(acc_u, u_hbm.at[pl.ds(base, rows_per)])
        pltpu.sync_copy(acc_d, den_hbm.at[pl.ds(base * L, rows_per * L)])

    return pl.kernel(
        body,
        out_type=[jax.ShapeDtypeStruct((n_pad, d), jnp.float32),
                  jax.ShapeDtypeStruct((n_pad * L,), jnp.float32)],
        mesh=mesh,
        compiler_params=pltpu.CompilerParams(needs_layout_passes=False, disable_bounds_checks=True),
        scratch_types=[
            pltpu.VMEM((rows_per, d), jnp.float32),
            pltpu.VMEM((rows_per * L,), jnp.float32),
            pltpu.VMEM((d,), jnp.float32),
            pltpu.VMEM((L,), jnp.float32),
            pltpu.VMEM((blk,), jnp.int32),
            pltpu.VMEM((blk,), jnp.int32),
            pltpu.VMEM((blk,), jnp.int32),
            pltpu.VMEM((blk,), jnp.int32),
            pltpu.VMEM((2 * blk + 3 * L,), jnp.int32),
            pltpu.VMEM((2 * blk + 3 * L,), jnp.int32),
            pltpu.VMEM((L, d), jnp.float32),
            pltpu.VMEM((L, d), jnp.float32),
            pltpu.VMEM((L, d), jnp.float32),
            pltpu.VMEM((L, d), jnp.float32),
            pltpu.SemaphoreType.DMA,
            pltpu.SemaphoreType.DMA,
            pltpu.SemaphoreType.DMA,
            pltpu.SemaphoreType.DMA,
            pltpu.SemaphoreType.DMA,
            pltpu.SemaphoreType.DMA,
        ],
    )


def kernel(ff, edge_index, W_edge, b_edge, W_attn, b_attn, W_node, b_node):
    n, d = ff.shape
    e = edge_index.shape[1]
    rows_per = (-(-n // NW) + 7) // 8 * 8
    blk = 1600
    assert e % blk == 0 and d % L == 0 and blk % L == 0

    w_cat = jnp.concatenate(
        [W_edge[:d, :], W_edge[d:, :], W_node[:d, :]], axis=1)
    b_cat = jnp.concatenate(
        [jnp.zeros((d,), jnp.float32), b_edge, b_node])[None, :]
    rb = 400
    grid = (n // rb,)
    P, Q, R = pl.pallas_call(
        _mm1_body,
        grid=grid,
        in_specs=[
            pl.BlockSpec((rb, d), lambda i: (i, 0)),
            pl.BlockSpec((d, 3 * d), lambda i: (0, 0)),
            pl.BlockSpec((1, 3 * d), lambda i: (0, 0)),
        ],
        out_specs=[
            pl.BlockSpec((rb, d), lambda i: (i, 0)),
            pl.BlockSpec((rb, d), lambda i: (i, 0)),
            pl.BlockSpec((rb, d), lambda i: (i, 0)),
        ],
        out_shape=[jax.ShapeDtypeStruct((n, d), jnp.float32)] * 3,
    )(ff, w_cat, b_cat)

    sc = _make_sc_kernel(n, e, d, rows_per, blk)
    wa = W_attn[:, 0]
    ba16 = jnp.full((L,), b_attn[0], jnp.float32)
    U, den_flat = sc(P, Q, edge_index[0], edge_index[1], wa, ba16)
    den = den_flat.reshape(-1, L)

    out = pl.pallas_call(
        _mm2_body,
        grid=grid,
        in_specs=[
            pl.BlockSpec((rb, d), lambda i: (i, 0)),
            pl.BlockSpec((rb, L), lambda i: (i, 0)),
            pl.BlockSpec((rb, d), lambda i: (i, 0)),
            pl.BlockSpec((d, d), lambda i: (0, 0)),
        ],
        out_specs=pl.BlockSpec((rb, d), lambda i: (i, 0)),
        out_shape=jax.ShapeDtypeStruct((n, d), jnp.float32),
    )(U[:n], den[:n], R, W_node[d:, :])
    return out


# speedup vs baseline: 6.3889x; 1.0383x over previous
import functools

import jax
import jax.numpy as jnp
from jax import lax
from jax.experimental import pallas as pl
from jax.experimental.pallas import tpu as pltpu
from jax.experimental.pallas import tpu_sc as plsc

NC = 2
NS = 16
NW = NC * NS
L = 16


def _mm1_body(ff_ref, w_ref, b_ref, p_ref, q_ref, r_ref):
    y = jnp.dot(ff_ref[...], w_ref[...], preferred_element_type=jnp.float32)
    y = y + b_ref[...]
    d = p_ref.shape[1]
    p_ref[...] = y[:, :d]
    q_ref[...] = y[:, d:2 * d]
    r_ref[...] = y[:, 2 * d:]


def _mm2_body(u_ref, den_ref, r_ref, w_ref, o_ref):
    agg = u_ref[...] / (den_ref[:, :1] + 1e-9)
    y = jnp.dot(agg, w_ref[...], preferred_element_type=jnp.float32)
    o_ref[...] = jnp.maximum(y + r_ref[...], 0.0)


def _make_sc_kernel(n, e, d, rows_per, blk):
    n_pad = NW * rows_per
    nchunk = d // L
    nscan = blk // L
    nblk = e // blk

    mesh = plsc.VectorSubcoreMesh(core_axis_name="c", subcore_axis_name="s")

    def body(p_hbm, q_hbm, src_hbm, dst_hbm, wa_hbm, ba_hbm,
             u_hbm, den_hbm,
             acc_u, acc_d, wa_v, ba_v, srcb, dstb, srcb2, dstb2, csrc, cdst,
             prows, qrows, prows2, qrows2, sem1, sem2, sem3, sem4, sem5, sem6):
        wid = lax.axis_index("s") * NC + lax.axis_index("c")
        base = wid * rows_per

        def zrow(r, _):
            def zch(ch, _):
                acc_u[r, pl.ds(ch * L, L)] = jnp.zeros((L,), jnp.float32)
                return 0
            lax.fori_loop(0, nchunk, zch, 0)
            acc_d[pl.ds(r * L, L)] = jnp.zeros((L,), jnp.float32)
            return 0
        lax.fori_loop(0, rows_per, zrow, 0)

        def zcomp(i, _):
            csrc[pl.ds(i * L, L)] = jnp.zeros((L,), jnp.int32)
            cdst[pl.ds(i * L, L)] = jnp.full((L,), base, jnp.int32)
            return 0
        lax.fori_loop(0, (4 * blk + 3 * L) // L, zcomp, 0)

        pltpu.sync_copy(wa_hbm, wa_v)
        pltpu.sync_copy(ba_hbm, ba_v)
        ba = ba_v[...]
        onehot0 = (lax.iota(jnp.int32, L) == 0).astype(jnp.float32)

        def fire_ids(b, sb, db, s):
            off = b * blk
            pltpu.async_copy(src_hbm.at[pl.ds(off, blk)], sb, s)
            pltpu.async_copy(dst_hbm.at[pl.ds(off, blk)], db, s)

        def wait_ids(sb, db, s):
            pltpu.make_async_copy(src_hbm.at[pl.ds(0, blk)], sb, s).wait()
            pltpu.make_async_copy(dst_hbm.at[pl.ds(0, blk)], db, s).wait()

        def scan_block(sb, db, cnt0):
            def scan(i, cnt):
                sv = sb[pl.ds(i * L, L)]
                dv = db[pl.ds(i * L, L)]
                msk = (dv >= base) & (dv < base + rows_per)
                plsc.store_compressed(csrc.at[pl.ds(cnt, L)], sv, mask=msk)
                plsc.store_compressed(cdst.at[pl.ds(cnt, L)], dv, mask=msk)
                return cnt + plsc.all_reduce_population_count(msk)[0]
            return lax.fori_loop(0, nscan, scan, cnt0, unroll=4)

        def fire_rows(g, pb, qb, s1, s2):
            pltpu.async_copy(p_hbm.at[csrc.at[pl.ds(g * L, L)]], pb, s1)
            pltpu.async_copy(q_hbm.at[cdst.at[pl.ds(g * L, L)]], qb, s2)

        def wait_rows(pb, qb, s1, s2):
            pltpu.make_async_copy(p_hbm.at[csrc.at[pl.ds(0, L)]],
                                  pb, s1).wait()
            pltpu.make_async_copy(q_hbm.at[cdst.at[pl.ds(0, L)]],
                                  qb, s2).wait()

        def process(g, cnt, pb, qb):
            def edge(i, _):
                j = g * L + i
                rloc = cdst[pl.ds(j, L)][0] - base
                ms = []
                acc = jnp.zeros((L,), jnp.float32)
                for ch in range(nchunk):
                    o = ch * L
                    m = jnp.maximum(pb[i, pl.ds(o, L)]
                                    + qb[i, pl.ds(o, L)], 0.0)
                    ms.append(m)
                    acc = acc + m * wa_v[pl.ds(o, L)]
                logit = jnp.sum(acc)
                valid = jnp.where(j < cnt, 1.0, 0.0)
                exv = jnp.exp(jnp.full((L,), logit, jnp.float32) + ba)
                exv = exv * valid
                for ch in range(nchunk):
                    plsc.addupdate(acc_u.at[rloc, pl.ds(ch * L, L)],
                                   exv * ms[ch])
                plsc.addupdate(acc_d.at[pl.ds(rloc * L, L)], exv * onehot0)
                return 0
            lax.fori_loop(0, L, edge, 0, unroll=2)

        def process_groups(cnt):
            ngroups = (cnt + (L - 1)) // L

            @pl.when(ngroups > 0)
            def _():
                fire_rows(0, prows, qrows, sem1, sem2)

            def pair(p, _):
                g0 = 2 * p

                @pl.when(g0 + 1 < ngroups)
                def _():
                    fire_rows(g0 + 1, prows2, qrows2, sem3, sem4)
                wait_rows(prows, qrows, sem1, sem2)
                process(g0, cnt, prows, qrows)

                @pl.when(g0 + 2 < ngroups)
                def _():
                    fire_rows(g0 + 2, prows, qrows, sem1, sem2)

                @pl.when(g0 + 1 < ngroups)
                def _():
                    wait_rows(prows2, qrows2, sem3, sem4)
                    process(g0 + 1, cnt, prows2, qrows2)
                return 0
            lax.fori_loop(0, (ngroups + 1) // 2, pair, 0)

        fire_ids(0, srcb, dstb, sem5)

        fire_ids(1, srcb2, dstb2, sem6)

        def blkquad(p, _):
            b0 = 4 * p
            wait_ids(srcb, dstb, sem5)
            cnt1 = scan_block(srcb, dstb, 0)
            wait_ids(srcb2, dstb2, sem6)
            cnt2 = scan_block(srcb2, dstb2, cnt1)
            fire_ids(b0 + 2, srcb, dstb, sem5)
            fire_ids(b0 + 3, srcb2, dstb2, sem6)
            wait_ids(srcb, dstb, sem5)
            cnt3 = scan_block(srcb, dstb, cnt2)
            wait_ids(srcb2, dstb2, sem6)
            cnt = scan_block(srcb2, dstb2, cnt3)

            @pl.when(b0 + 4 < nblk)
            def _():
                fire_ids(b0 + 4, srcb, dstb, sem5)
                fire_ids(b0 + 5, srcb2, dstb2, sem6)
            process_groups(cnt)
            return 0
        lax.fori_loop(0, nblk // 4, blkquad, 0)

        pltpu.sync_copy(acc_u, u_hbm.at[pl.ds(base, rows_per)])
        pltpu.sync_copy(acc_d, den_hbm.at[pl.ds(base * L, rows_per * L)])

    return pl.kernel(
        body,
        out_type=[jax.ShapeDtypeStruct((n_pad, d), jnp.float32),
                  jax.ShapeDtypeStruct((n_pad * L,), jnp.float32)],
        mesh=mesh,
        compiler_params=pltpu.CompilerParams(needs_layout_passes=False, disable_bounds_checks=True),
        scratch_types=[
            pltpu.VMEM((rows_per, d), jnp.float32),
            pltpu.VMEM((rows_per * L,), jnp.float32),
            pltpu.VMEM((d,), jnp.float32),
            pltpu.VMEM((L,), jnp.float32),
            pltpu.VMEM((blk,), jnp.int32),
            pltpu.VMEM((blk,), jnp.int32),
            pltpu.VMEM((blk,), jnp.int32),
            pltpu.VMEM((blk,), jnp.int32),
            pltpu.VMEM((4 * blk + 3 * L,), jnp.int32),
            pltpu.VMEM((4 * blk + 3 * L,), jnp.int32),
            pltpu.VMEM((L, d), jnp.float32),
            pltpu.VMEM((L, d), jnp.float32),
            pltpu.VMEM((L, d), jnp.float32),
            pltpu.VMEM((L, d), jnp.float32),
            pltpu.SemaphoreType.DMA,
            pltpu.SemaphoreType.DMA,
            pltpu.SemaphoreType.DMA,
            pltpu.SemaphoreType.DMA,
            pltpu.SemaphoreType.DMA,
            pltpu.SemaphoreType.DMA,
        ],
    )


def kernel(ff, edge_index, W_edge, b_edge, W_attn, b_attn, W_node, b_node):
    n, d = ff.shape
    e = edge_index.shape[1]
    rows_per = (-(-n // NW) + 7) // 8 * 8
    blk = 1600
    assert e % blk == 0 and (e // blk) % 4 == 0 and d % L == 0 and blk % L == 0

    w_cat = jnp.concatenate(
        [W_edge[:d, :], W_edge[d:, :], W_node[:d, :]], axis=1)
    b_cat = jnp.concatenate(
        [jnp.zeros((d,), jnp.float32), b_edge, b_node])[None, :]
    rb = 400
    grid = (n // rb,)
    P, Q, R = pl.pallas_call(
        _mm1_body,
        grid=grid,
        in_specs=[
            pl.BlockSpec((rb, d), lambda i: (i, 0)),
            pl.BlockSpec((d, 3 * d), lambda i: (0, 0)),
            pl.BlockSpec((1, 3 * d), lambda i: (0, 0)),
        ],
        out_specs=[
            pl.BlockSpec((rb, d), lambda i: (i, 0)),
            pl.BlockSpec((rb, d), lambda i: (i, 0)),
            pl.BlockSpec((rb, d), lambda i: (i, 0)),
        ],
        out_shape=[jax.ShapeDtypeStruct((n, d), jnp.float32)] * 3,
    )(ff, w_cat, b_cat)

    sc = _make_sc_kernel(n, e, d, rows_per, blk)
    wa = W_attn[:, 0]
    ba16 = jnp.full((L,), b_attn[0], jnp.float32)
    U, den_flat = sc(P, Q, edge_index[0], edge_index[1], wa, ba16)
    den = den_flat.reshape(-1, L)

    out = pl.pallas_call(
        _mm2_body,
        grid=grid,
        in_specs=[
            pl.BlockSpec((rb, d), lambda i: (i, 0)),
            pl.BlockSpec((rb, L), lambda i: (i, 0)),
            pl.BlockSpec((rb, d), lambda i: (i, 0)),
            pl.BlockSpec((d, d), lambda i: (0, 0)),
        ],
        out_specs=pl.BlockSpec((rb, d), lambda i: (i, 0)),
        out_shape=jax.ShapeDtypeStruct((n, d), jnp.float32),
    )(U[:n], den[:n], R, W_node[d:, :])
    return out
